# baseline (speedup 1.0000x reference)
import dataclasses
import functools

import jax
import jax.numpy as jnp
from jax import lax
from jax.experimental import pallas as pl
from jax.experimental.pallas import tpu as pltpu
from jax.experimental.pallas import tpu_sc as plsc

_B, _C, _S, _H = 1024, 26, 50, 32
_M = _B * _C * _S
_NC, _NS = 2, 16
_NW = _NC * _NS
_GROUP = 128
_GPC = 8
_CHUNK = _GPC * _GROUP
_NPAIRS = _C * _S
_B4 = _B // 4
_KSLICES = 2
_CSLICE = _C // _KSLICES


def _sc_gather(idx2d, word_table, pair_lo, npairs):
    mesh = plsc.VectorSubcoreMesh(core_axis_name="c", subcore_axis_name="s")

    @functools.partial(
        pl.kernel,
        mesh=mesh,
        out_type=jax.ShapeDtypeStruct((npairs * _CHUNK, _H), jnp.float32),
        compiler_params=pltpu.CompilerParams(use_tc_tiling_on_sc=False),
        scratch_types=[
            pltpu.VMEM((_GPC, _GROUP), jnp.int32),
            pltpu.VMEM((_CHUNK, _H), jnp.float32),
            pltpu.SemaphoreType.DMA,
        ],
    )
    def k(idx_hbm, table_hbm, out_hbm, idx_v, rows_v, gsem):
        wid = lax.axis_index("s") * _NC + lax.axis_index("c")
        p_start = (npairs * wid) // _NW
        p_end = (npairs * (wid + 1)) // _NW

        @pl.loop(0, p_end - p_start)
        def _(i):
            p = p_start + i
            pg = pair_lo + p
            pltpu.sync_copy(idx_hbm.at[pg], idx_v)
            copies = []
            for j in range(_GPC):
                copies.append(
                    pltpu.async_copy(
                        table_hbm.at[idx_v.at[j]],
                        rows_v.at[pl.ds(j * _GROUP, _GROUP)],
                        gsem,
                    )
                )
            for cp in copies:
                cp.wait()
            row0 = pl.multiple_of(p * _CHUNK, 1024)
            pltpu.sync_copy(rows_v, out_hbm.at[pl.ds(row0, _CHUNK)])

    return k(idx2d, word_table)


_NSB = 25
_RB = _NSB * _B4
_BW = _B // _NW


def _sc_idx_format(x):
    mesh = plsc.VectorSubcoreMesh(core_axis_name="c", subcore_axis_name="s")
    cp = pltpu.CompilerParams(use_tc_tiling_on_sc=False)
    if "needs_layout_passes" in pltpu.CompilerParams.__dataclass_fields__:
        cp = dataclasses.replace(cp, needs_layout_passes=False)

    @functools.partial(
        pl.kernel,
        mesh=mesh,
        out_type=jax.ShapeDtypeStruct((_NPAIRS, _GPC, _GROUP), jnp.int32),
        compiler_params=cp,
        scratch_types=[
            pltpu.VMEM((_BW, _C, _S), jnp.int32),
            pltpu.VMEM((_NPAIRS, _BW), jnp.int32),
        ],
    )
    def k(x_hbm, out_hbm, slab_v, idxt_v):
        wid = lax.axis_index("s") * _NC + lax.axis_index("c")
        pltpu.sync_copy(x_hbm.at[pl.ds(wid * _BW, _BW)], slab_v)
        b_lo = lax.iota(jnp.int32, 16)
        b_hi = b_lo + 16

        @pl.loop(0, _NPAIRS)
        def _(p):
            c = jnp.full((16,), p // _S, jnp.int32)
            s = jnp.full((16,), p % _S, jnp.int32)
            idxt_v[p, pl.ds(0, 16)] = plsc.load_gather(slab_v, [b_lo, c, s])
            idxt_v[p, pl.ds(16, 16)] = plsc.load_gather(slab_v, [b_hi, c, s])

        pltpu.sync_copy(
            idxt_v,
            out_hbm.at[:, wid // 4, pl.ds((wid % 4) * _BW, _BW)],
        )

    return k(x)


def _tc_body(weg_ref, bias_ref, wrow_ref, gamma_ref, beta_ref, G_ref,
             prev_ref, o_ref):
    c = pl.program_id(1)
    X = weg_ref[...]
    bexp = jnp.broadcast_to(
        bias_ref[0, :, 0, :][:, None, :], (_NSB, _B4, 128)).reshape(_RB, 128)
    wexp = jnp.broadcast_to(
        wrow_ref[0, :, 0, :][:, None, :], (_NSB, _B4, 128)).reshape(_RB, 128)
    emb = X + bexp
    G = G_ref[...]
    s1 = jax.lax.dot(emb, G)
    s2 = jax.lax.dot(emb * emb, G)
    mu = s1 * (1.0 / _H)
    var = s2 * (1.0 / _H) - mu * mu
    rstd = jax.lax.rsqrt(var + 1e-5)
    contrib = (emb - mu) * (rstd * wexp * gamma_ref[...]) \
        + wexp * beta_ref[...]
    contrib = contrib.reshape(_NSB, _B4, 128)

    @pl.when(c == 0)
    def _():
        o_ref[...] = prev_ref[...] + contrib

    @pl.when(c > 0)
    def _():
        o_ref[...] += contrib


def _tc_fuse(weg2, wrow, bias, gamma128, beta128, G, prev, ncs):
    return pl.pallas_call(
        _tc_body,
        grid=(_S // _NSB, ncs),
        in_specs=[
            pl.BlockSpec((_RB, 128),
                         lambda sb, c: (c * (_S // _NSB) + sb, 0)),
            pl.BlockSpec((1, _NSB, 1, 128), lambda sb, c: (c, sb, 0, 0)),
            pl.BlockSpec((1, _NSB, 1, 128), lambda sb, c: (c, sb, 0, 0)),
            pl.BlockSpec((1, 128), lambda sb, c: (0, 0)),
            pl.BlockSpec((1, 128), lambda sb, c: (0, 0)),
            pl.BlockSpec((128, 128), lambda sb, c: (0, 0)),
            pl.BlockSpec((_NSB, _B4, 128), lambda sb, c: (sb, 0, 0)),
        ],
        out_specs=pl.BlockSpec((_NSB, _B4, 128), lambda sb, c: (sb, 0, 0)),
        out_shape=jax.ShapeDtypeStruct((_S, _B4, 128), jnp.float32),
        input_output_aliases={6: 0},
    )(weg2, bias, wrow, gamma128, beta128, G, prev)


def kernel(x, word_table, pos_table, ch_table, ln_gamma, ln_beta, fusion_w):
    w = jax.nn.softmax(fusion_w, axis=-1)
    wrow = jnp.broadcast_to(w.T[:, :, None], (_C, _S, 128))
    wrow = wrow.reshape(_C, _S, 1, 128)
    bias = ch_table[:, None, :] + pos_table[None, :, :]
    bias = jnp.tile(bias, (1, 1, 4)).reshape(_C, _S, 1, 128)
    gamma128 = jnp.tile(ln_gamma, 4).reshape(1, 128)
    beta128 = jnp.tile(ln_beta, 4).reshape(1, 128)
    gi = jax.lax.broadcasted_iota(jnp.int32, (128, 128), 0) // _H
    gj = jax.lax.broadcasted_iota(jnp.int32, (128, 128), 1) // _H
    G = (gi == gj).astype(jnp.float32)

    idx4 = _sc_idx_format(x)

    out4 = jnp.zeros((_S, _B4, 128), jnp.float32)
    npairs = _CSLICE * _S
    for k in range(_KSLICES):
        c_lo = k * _CSLICE
        weg = _sc_gather(idx4, word_table, c_lo * _S, npairs)
        weg2 = weg.reshape(npairs * _CHUNK // 4, 128)
        out4 = _tc_fuse(weg2, wrow[c_lo:c_lo + _CSLICE],
                        bias[c_lo:c_lo + _CSLICE], gamma128, beta128, G,
                        out4, _CSLICE)
    return jnp.transpose(out4.reshape(_S, _B, _H), (1, 0, 2))


# speedup vs baseline: 1.0160x; 1.0160x over previous
import dataclasses
import functools

import jax
import jax.numpy as jnp
from jax import lax
from jax.experimental import pallas as pl
from jax.experimental.pallas import tpu as pltpu
from jax.experimental.pallas import tpu_sc as plsc

_B, _C, _S, _H = 1024, 26, 50, 32
_M = _B * _C * _S
_NC, _NS = 2, 16
_NW = _NC * _NS
_GROUP = 128
_GPC = 8
_CHUNK = _GPC * _GROUP
_NPAIRS = _C * _S
_B4 = _B // 4
_KSLICES = 2
_CSLICE = _C // _KSLICES


def _sc_gather(idx2d, word_table, pair_lo, npairs):
    mesh = plsc.VectorSubcoreMesh(core_axis_name="c", subcore_axis_name="s")

    @functools.partial(
        pl.kernel,
        mesh=mesh,
        out_type=jax.ShapeDtypeStruct((npairs * _CHUNK, _H), jnp.float32),
        compiler_params=pltpu.CompilerParams(use_tc_tiling_on_sc=False),
        scratch_types=[
            pltpu.VMEM((_GPC, _GROUP), jnp.int32),
            pltpu.VMEM((_CHUNK, _H), jnp.float32),
            pltpu.SemaphoreType.DMA,
        ],
    )
    def k(idx_hbm, table_hbm, out_hbm, idx_v, rows_v, gsem):
        wid = lax.axis_index("s") * _NC + lax.axis_index("c")
        p_start = (npairs * wid) // _NW
        p_end = (npairs * (wid + 1)) // _NW

        @pl.loop(0, p_end - p_start)
        def _(i):
            p = p_start + i
            pg = pair_lo + p
            pltpu.sync_copy(idx_hbm.at[pg], idx_v)
            copies = []
            for j in range(_GPC):
                copies.append(
                    pltpu.async_copy(
                        table_hbm.at[idx_v.at[j]],
                        rows_v.at[pl.ds(j * _GROUP, _GROUP)],
                        gsem,
                    )
                )
            for cp in copies:
                cp.wait()
            row0 = pl.multiple_of(p * _CHUNK, 1024)
            pltpu.sync_copy(rows_v, out_hbm.at[pl.ds(row0, _CHUNK)])

    return k(idx2d, word_table)


_NSB = 25
_RB = _NSB * _B4
_BW = _B // _NW


def _sc_idx_format(x):
    mesh = plsc.VectorSubcoreMesh(core_axis_name="c", subcore_axis_name="s")
    cp = pltpu.CompilerParams(use_tc_tiling_on_sc=False)
    if "needs_layout_passes" in pltpu.CompilerParams.__dataclass_fields__:
        cp = dataclasses.replace(cp, needs_layout_passes=False)

    @functools.partial(
        pl.kernel,
        mesh=mesh,
        out_type=jax.ShapeDtypeStruct((_NPAIRS, _GPC, _GROUP), jnp.int32),
        compiler_params=cp,
        scratch_types=[
            pltpu.VMEM((_BW, _C, _S), jnp.int32),
            pltpu.VMEM((_NPAIRS, _BW), jnp.int32),
        ],
    )
    def k(x_hbm, out_hbm, slab_v, idxt_v):
        wid = lax.axis_index("s") * _NC + lax.axis_index("c")
        pltpu.sync_copy(x_hbm.at[pl.ds(wid * _BW, _BW)], slab_v)
        b_lo = lax.iota(jnp.int32, 16)
        b_hi = b_lo + 16

        @pl.loop(0, _NPAIRS)
        def _(p):
            c = jnp.full((16,), p // _S, jnp.int32)
            s = jnp.full((16,), p % _S, jnp.int32)
            idxt_v[p, pl.ds(0, 16)] = plsc.load_gather(slab_v, [b_lo, c, s])
            idxt_v[p, pl.ds(16, 16)] = plsc.load_gather(slab_v, [b_hi, c, s])

        pltpu.sync_copy(
            idxt_v,
            out_hbm.at[:, wid // 4, pl.ds((wid % 4) * _BW, _BW)],
        )

    return k(x)


def _tc_body(weg_ref, bias_ref, wrow_ref, gamma_ref, beta_ref, G_ref,
             prev_ref, o_ref):
    c = pl.program_id(1)
    X = weg_ref[...]
    bexp = jnp.broadcast_to(
        bias_ref[0, :, 0, :][:, None, :], (_NSB, _B4, 128)).reshape(_RB, 128)
    wexp = jnp.broadcast_to(
        wrow_ref[0, :, 0, :][:, None, :], (_NSB, _B4, 128)).reshape(_RB, 128)
    emb = X + bexp
    G = G_ref[...]
    s1 = jax.lax.dot(emb, G)
    s2 = jax.lax.dot(emb * emb, G)
    mu = s1 * (1.0 / _H)
    var = s2 * (1.0 / _H) - mu * mu
    rstd = jax.lax.rsqrt(var + 1e-5)
    contrib = (emb - mu) * (rstd * wexp * gamma_ref[...]) \
        + wexp * beta_ref[...]
    contrib = contrib.reshape(_NSB, _B4, 128)

    @pl.when(c == 0)
    def _():
        o_ref[...] = prev_ref[...] + contrib

    @pl.when(c > 0)
    def _():
        o_ref[...] += contrib


def _tc_fuse(weg2, wrow, bias, gamma128, beta128, G, prev, ncs):
    return pl.pallas_call(
        _tc_body,
        grid=(_S // _NSB, ncs),
        in_specs=[
            pl.BlockSpec((_RB, 128),
                         lambda sb, c: (c * (_S // _NSB) + sb, 0)),
            pl.BlockSpec((1, _NSB, 1, 128), lambda sb, c: (c, sb, 0, 0)),
            pl.BlockSpec((1, _NSB, 1, 128), lambda sb, c: (c, sb, 0, 0)),
            pl.BlockSpec((1, 128), lambda sb, c: (0, 0)),
            pl.BlockSpec((1, 128), lambda sb, c: (0, 0)),
            pl.BlockSpec((128, 128), lambda sb, c: (0, 0)),
            pl.BlockSpec((_NSB, _B4, 128), lambda sb, c: (sb, 0, 0)),
        ],
        out_specs=pl.BlockSpec((_NSB, _B4, 128), lambda sb, c: (sb, 0, 0)),
        out_shape=jax.ShapeDtypeStruct((_S, _B4, 128), jnp.float32),
        input_output_aliases={6: 0},
    )(weg2, bias, wrow, gamma128, beta128, G, prev)


def kernel(x, word_table, pos_table, ch_table, ln_gamma, ln_beta, fusion_w):
    w = jax.nn.softmax(fusion_w, axis=-1)
    wrow = jnp.broadcast_to(w.T[:, :, None], (_C, _S, 128))
    wrow = wrow.reshape(_C, _S, 1, 128)
    bias = ch_table[:, None, :] + pos_table[None, :, :]
    bias = jnp.tile(bias, (1, 1, 4)).reshape(_C, _S, 1, 128)
    gamma128 = jnp.tile(ln_gamma, 4).reshape(1, 128)
    beta128 = jnp.tile(ln_beta, 4).reshape(1, 128)
    gi = jax.lax.broadcasted_iota(jnp.int32, (128, 128), 0) // _H
    gj = jax.lax.broadcasted_iota(jnp.int32, (128, 128), 1) // _H
    G = (gi == gj).astype(jnp.float32)

    idx4 = jnp.transpose(x, (1, 2, 0)).reshape(_NPAIRS, _GPC, _GROUP)

    out4 = jnp.zeros((_S, _B4, 128), jnp.float32)
    npairs = _CSLICE * _S
    for k in range(_KSLICES):
        c_lo = k * _CSLICE
        weg = _sc_gather(idx4, word_table, c_lo * _S, npairs)
        weg2 = weg.reshape(npairs * _CHUNK // 4, 128)
        out4 = _tc_fuse(weg2, wrow[c_lo:c_lo + _CSLICE],
                        bias[c_lo:c_lo + _CSLICE], gamma128, beta128, G,
                        out4, _CSLICE)
    return jnp.transpose(out4.reshape(_S, _B, _H), (1, 0, 2))
